# baseline (speedup 1.0000x reference)
import functools
import math

import jax
import jax.numpy as jnp
from jax import lax
from jax.experimental import pallas as pl
from jax.experimental.pallas import tpu as pltpu
from jax.experimental.pallas import tpu_sc as plsc

B = 4096
L = 200
D = 64
DP = 128
V = 1000000
SCALE = math.sqrt(D)

NW = 32
ROWS = B * L
PER_W = ROWS // NW
BROWS_W = B // NW
TBLK = 32
LANES = 16

PK = 64
PCHUNKS = V // PK
PROUNDS = -(-PCHUNKS // NW)

HOFF = (0, 72, 136)
HSZ = (72, 64, 64)
HMAX = 72


VP = 1000064
NCH = VP // 128
PT_ROUNDS = -(-NCH // NW)


def _padt_body(tabt_hbm, tpad_hbm, ins, stages, sem_i, sem_o):
    cid = lax.axis_index("c")
    sid = lax.axis_index("s")
    wid = sid * 2 + cid

    def col0_of(g):
        return g * 128

    def start_in(buf, g):
        pltpu.make_async_copy(
            tabt_hbm.at[:, pl.ds(col0_of(g), 128)], ins[buf], sem_i.at[buf]
        ).start()

    def wait_in(buf, g):
        pltpu.make_async_copy(
            tabt_hbm.at[:, pl.ds(col0_of(g), 128)], ins[buf], sem_i.at[buf]
        ).wait()

    def out_dma(buf, g):
        return pltpu.make_async_copy(
            stages[buf], tpad_hbm.at[pl.ds(col0_of(g), 128)], sem_o.at[buf]
        )

    def move(buf):
        src, dst = ins[buf], stages[buf]
        lane = lax.iota(jnp.int32, 16)

        def row(r, carry):
            col = jnp.zeros((16,), jnp.int32) + r
            for j in range(D // LANES):
                vals = plsc.load_gather(src, [lane + j * LANES, col])
                dst[r, pl.ds(j * LANES, LANES)] = vals
            return carry

        lax.fori_loop(0, 128, row, 0)

    for buf in range(2):
        @pl.when(buf * NW + wid < NCH)
        def _(buf=buf):
            start_in(buf, buf * NW + wid)

    def outer(t, carry):
        for buf in range(2):
            rnd = t * 2 + buf
            g = rnd * NW + wid

            @pl.when(g < NCH)
            def _():
                wait_in(buf, g)
                @pl.when(rnd >= 2)
                def _():
                    out_dma(buf, g - 2 * NW).wait()

                move(buf)

                @pl.when(g + 2 * NW < NCH)
                def _():
                    start_in(buf, g + 2 * NW)

                out_dma(buf, g).start()
        return carry

    lax.fori_loop(0, (PT_ROUNDS + 1) // 2, outer, 0)

    nr = jnp.where(wid < NCH - (PT_ROUNDS - 1) * NW, PT_ROUNDS, PT_ROUNDS - 1)
    for buf in range(2):
        last = nr - 1 - lax.rem(nr - 1 - buf + 2, 2)
        @pl.when(last >= 0)
        def _(buf=buf, last=last):
            out_dma(buf, last * NW + wid).wait()


def _gather_body(tok_hbm, tpad_hbm, out_hbm, tok_v, idx_v, ins, outs,
                 sem_g, sem_s):
    cid = lax.axis_index("c")
    sid = lax.axis_index("s")
    wid = sid * 2 + cid
    bbase = wid * BROWS_W

    offs = [16 * k for k in range(12)] + [184]

    for blk in range(BROWS_W // TBLK):
        pltpu.sync_copy(tok_hbm.at[pl.ds(bbase + blk * TBLK, TBLK)], tok_v)

        def row(r, carry, blk=blk):
            for o in offs:
                idx_v[pl.ds((blk * TBLK + r) * L + o, LANES)] = (
                    tok_v[r, pl.ds(o, LANES)]
                )
            return carry

        lax.fori_loop(0, TBLK, row, 0)

    def start_gather(h, bb):
        pltpu.make_async_copy(
            tpad_hbm.at[idx_v.at[pl.ds(bb * L + HOFF[h], HSZ[h])]],
            ins[h].at[pl.ds(0, HSZ[h])],
            sem_g.at[h],
        ).start()

    def wait_gather(h, bb):
        pltpu.make_async_copy(
            tpad_hbm.at[idx_v.at[pl.ds(bb * L + HOFF[h], HSZ[h])]],
            ins[h].at[pl.ds(0, HSZ[h])],
            sem_g.at[h],
        ).wait()

    def out_dma(h, bb):
        return pltpu.make_async_copy(
            outs[h].at[pl.ds(0, HSZ[h])],
            out_hbm.at[bbase + bb, pl.ds(HOFF[h], HSZ[h])],
            sem_s.at[h],
        )

    def scale(h):
        src, dst = ins[h], outs[h]

        def rowblk(i, carry):
            r0 = i * 8
            for rr in range(8):
                for j in range(D // LANES):
                    sl = pl.ds(j * LANES, LANES)
                    dst[r0 + rr, sl] = src[r0 + rr, sl] * jnp.float32(SCALE)
            return carry

        lax.fori_loop(0, HSZ[h] // 8, rowblk, 0)

    for h in range(3):
        start_gather(h, 0)

    def outer(bb, carry):
        for h in range(3):
            wait_gather(h, bb)
            @pl.when(bb >= 1)
            def _():
                out_dma(h, bb - 1).wait()

            scale(h)

            @pl.when(bb + 1 < BROWS_W)
            def _():
                start_gather(h, bb + 1)

            out_dma(h, bb).start()
        return carry

    lax.fori_loop(0, BROWS_W, outer, 0)

    for h in range(3):
        out_dma(h, BROWS_W - 1).wait()


_padt = functools.partial(
    pl.kernel,
    mesh=plsc.VectorSubcoreMesh(core_axis_name="c", subcore_axis_name="s"),
    out_type=jax.ShapeDtypeStruct((VP, DP), jnp.float32),
    scratch_types=[
        [pltpu.VMEM((D, 128), jnp.float32) for _ in range(2)],
        [pltpu.VMEM((128, DP), jnp.float32) for _ in range(2)],
        pltpu.SemaphoreType.DMA((2,)),
        pltpu.SemaphoreType.DMA((2,)),
    ],
    compiler_params=pltpu.CompilerParams(needs_layout_passes=False),
)(_padt_body)


_gather = functools.partial(
    pl.kernel,
    mesh=plsc.VectorSubcoreMesh(core_axis_name="c", subcore_axis_name="s"),
    out_type=jax.ShapeDtypeStruct((B, L, D), jnp.float32),
    scratch_types=[
        pltpu.VMEM((TBLK, L), jnp.int32),
        pltpu.VMEM((PER_W,), jnp.int32),
        [pltpu.VMEM((HMAX, DP), jnp.float32) for _ in range(3)],
        [pltpu.VMEM((HMAX, D), jnp.float32) for _ in range(3)],
        pltpu.SemaphoreType.DMA((3,)),
        pltpu.SemaphoreType.DMA((3,)),
    ],
)(_gather_body)


def kernel(tokens, table):
    tpad = _padt(jnp.pad(table, ((0, VP - V), (0, 0))).T)
    return _gather(tokens, tpad)


# speedup vs baseline: 2.1977x; 2.1977x over previous
import functools
import math

import jax
import jax.numpy as jnp
from jax import lax
from jax.experimental import pallas as pl
from jax.experimental.pallas import tpu as pltpu
from jax.experimental.pallas import tpu_sc as plsc

B = 4096
L = 200
D = 64
DP = 128
SCALE = math.sqrt(D)

NW = 32
ROWS = B * L
PER_W = ROWS // NW
BROWS_W = B // NW
TBLK = 32
C = 80
G = PER_W // C
NBUF = 4
LANES = 16
RUNROLL = 8


def _sc_body(tok_hbm, table_hbm, out_hbm, tok_v, idx_v, ins, outs,
             sem_g, sem_s):
    cid = lax.axis_index("c")
    sid = lax.axis_index("s")
    wid = sid * 2 + cid
    bbase = wid * BROWS_W
    base = wid * PER_W

    offs = [16 * k for k in range(12)] + [184]

    for blk in range(BROWS_W // TBLK):
        pltpu.sync_copy(tok_hbm.at[pl.ds(bbase + blk * TBLK, TBLK)], tok_v)

        def row(r, carry, blk=blk):
            for o in offs:
                idx_v[pl.ds((blk * TBLK + r) * L + o, LANES)] = (
                    tok_v[r, pl.ds(o, LANES)]
                )
            return carry

        lax.fori_loop(0, TBLK, row, 0)

    def start_gather(buf, g):
        pltpu.make_async_copy(
            table_hbm.at[idx_v.at[pl.ds(g * C, C)]], ins[buf], sem_g.at[buf]
        ).start()

    def wait_gather(buf, g):
        pltpu.make_async_copy(
            table_hbm.at[idx_v.at[pl.ds(g * C, C)]], ins[buf], sem_g.at[buf]
        ).wait()

    def scale(buf):
        src, dst = ins[buf], outs[buf]

        def rowblk(i, carry):
            r0 = i * RUNROLL
            for rr in range(RUNROLL):
                for j in range(D // LANES):
                    sl = pl.ds(j * LANES, LANES)
                    dst[r0 + rr, sl] = src[r0 + rr, sl] * jnp.float32(SCALE)
            return carry

        lax.fori_loop(0, C // RUNROLL, rowblk, 0)

    for buf in range(NBUF):
        start_gather(buf, buf)

    def outer(t, carry):
        for buf in range(NBUF):
            g = t * NBUF + buf
            wait_gather(buf, g)
            @pl.when(g >= NBUF)
            def _():
                pltpu.make_async_copy(
                    outs[buf],
                    out_hbm.at[pl.ds(base + (g - NBUF) * C, C)],
                    sem_s.at[buf],
                ).wait()

            scale(buf)

            @pl.when(g + NBUF < G)
            def _():
                start_gather(buf, g + NBUF)

            pltpu.make_async_copy(
                outs[buf], out_hbm.at[pl.ds(base + g * C, C)], sem_s.at[buf]
            ).start()
        return carry

    lax.fori_loop(0, G // NBUF, outer, 0)

    for buf in range(NBUF):
        g = G - NBUF + buf
        pltpu.make_async_copy(
            outs[buf], out_hbm.at[pl.ds(base + g * C, C)], sem_s.at[buf]
        ).wait()


_sc_gather = functools.partial(
    pl.kernel,
    mesh=plsc.VectorSubcoreMesh(core_axis_name="c", subcore_axis_name="s"),
    out_type=jax.ShapeDtypeStruct((ROWS, D), jnp.float32),
    scratch_types=[
        pltpu.VMEM((TBLK, L), jnp.int32),
        pltpu.VMEM((PER_W,), jnp.int32),
        [pltpu.VMEM((C, DP), jnp.float32) for _ in range(NBUF)],
        [pltpu.VMEM((C, D), jnp.float32) for _ in range(NBUF)],
        pltpu.SemaphoreType.DMA((NBUF,)),
        pltpu.SemaphoreType.DMA((NBUF,)),
    ],
)(_sc_body)


def kernel(tokens, table):
    tpad = jnp.pad(table, ((0, 0), (0, DP - D)))
    out = _sc_gather(tokens, tpad)
    return out.reshape(B, L, D)
